# Initial kernel scaffold; baseline (speedup 1.0000x reference)
import jax
import jax.numpy as jnp
from jax.experimental import pallas as pl


def kernel(x, t1_w, t1_b, t2_w, t2_b, t3_w, t3_b, s1_w, s1_b, s2_w, s2_b, s3_w, s3_b, c_w, c_b):
    raise NotImplementedError("write your pallas kernel here")


# speedup vs baseline: 7.6024x; 7.6024x over previous
import functools

import numpy as np
import jax
import jax.numpy as jnp
from jax import lax
from jax.experimental import pallas as pl
from jax.experimental.pallas import tpu as pltpu
from jax.experimental.pallas import tpu_sc as plsc

IN_CH = 256
OUT_CH = 128
TSCALE = 128
DSCALE = 32
KNN_K = 3
ROI_SIZE = 32
CTX_SIZE = 4
PROP_EXT = 0.5
GROUPS = 32
WIDTH = 4 * GROUPS
NSLOT = ROI_SIZE + CTX_SIZE
NANCH = DSCALE * TSCALE
NTAP = 2 * NSLOT


def _align_consts():
    anchors = []
    for dur_idx in range(DSCALE):
        for start_idx in range(TSCALE):
            end_idx = start_idx + dur_idx + 1
            if end_idx <= TSCALE:
                cl = float(dur_idx + 1)
                anchors.append([start_idx - cl * PROP_EXT, end_idx + cl * PROP_EXT])
            else:
                anchors.append([0.0, 0.0])
    anchors = np.asarray(anchors, dtype=np.float32)

    idx_list, wgt_list = [], []
    for out_size, slot0 in ((ROI_SIZE, 0), (CTX_SIZE, ROI_SIZE)):
        start, end = anchors[:, 0], anchors[:, 1]
        length = np.maximum(end - start, 1.0)
        bin_size = length / float(out_size)
        i = np.arange(out_size, dtype=np.float32)
        pos = start[:, None] + bin_size[:, None] * (i[None, :] + 0.5)
        valid = (pos >= -1.0) & (pos <= float(TSCALE))
        p = np.clip(pos, 0.0, float(TSCALE - 1))
        lo = np.floor(p).astype(np.int32)
        hi = np.minimum(lo + 1, TSCALE - 1)
        w_hi = (p - lo.astype(np.float32)) * valid
        w_lo = (1.0 - (p - lo.astype(np.float32))) * valid
        slot = slot0 + np.arange(out_size, dtype=np.int32)[None, :]
        idx_list += [lo * NSLOT + slot, hi * NSLOT + slot]
        wgt_list += [w_lo, w_hi]
    idx = np.concatenate(idx_list, axis=1).astype(np.int32)
    wgt = np.concatenate(wgt_list, axis=1).astype(np.float32)
    return idx, wgt


_IDX_NP, _WGT_NP = _align_consts()
_WGTB_NP = np.repeat(_WGT_NP.reshape(-1, 1), 16, axis=1)


def _block_diag_dense(w):
    blocks = w.reshape(GROUPS, 4, 4).transpose(0, 2, 1)
    dense = jnp.zeros((GROUPS, 4, GROUPS, 4), jnp.float32)
    dense = dense.at[jnp.arange(GROUPS), :, jnp.arange(GROUPS), :].set(blocks)
    return dense.reshape(WIDTH, WIDTH)


def _dot(a, b):
    return jnp.dot(a, b, preferred_element_type=jnp.float32)


def _knn3_onehots(z):
    inner = _dot(z, z.T)
    xx = jnp.sum(z * z, axis=1)
    pd = 2.0 * inner - xx[None, :] - xx[:, None]
    iota = lax.broadcasted_iota(jnp.int32, (TSCALE, TSCALE), 1)
    ohs = []
    for _ in range(KNN_K):
        m = jnp.max(pd, axis=1, keepdims=True)
        idx = jnp.min(jnp.where(pd == m, iota, TSCALE), axis=1, keepdims=True)
        sel = iota == idx
        ohs.append(sel.astype(jnp.float32))
        pd = jnp.where(sel, -jnp.inf, pd)
    return ohs


def _tc_body(xt_ref, w1t_ref, b1_ref, d2m_ref, d2c_ref, d2p_ref, b2_ref,
             w3t_ref, b3_ref, w1at_ref, w1bt_ref, bs1_ref, ds2_ref, bs2_ref,
             ws3t_ref, bs3_ref, wi_ref, wctx_ref, out_ref):
    x = xt_ref[...]

    a1 = jax.nn.relu(_dot(x, w1t_ref[...]) + b1_ref[...])
    zrow = jnp.zeros((1, WIDTH), jnp.float32)
    a1m = jnp.concatenate([zrow, a1[:-1, :]], axis=0)
    a1p = jnp.concatenate([a1[1:, :], zrow], axis=0)
    a2 = jax.nn.relu(_dot(a1m, d2m_ref[...]) + _dot(a1, d2c_ref[...])
                     + _dot(a1p, d2p_ref[...]) + b2_ref[...])
    a3 = _dot(a2, w3t_ref[...]) + b3_ref[...]

    smax = None
    for oh in _knn3_onehots(x):
        fk = _dot(oh, x)
        b1k = jax.nn.relu(_dot(fk, w1at_ref[...]) + _dot(x, w1bt_ref[...])
                          + bs1_ref[...])
        b2k = jax.nn.relu(_dot(b1k, ds2_ref[...]) + bs2_ref[...])
        b3k = _dot(b2k, ws3t_ref[...]) + bs3_ref[...]
        smax = b3k if smax is None else jnp.maximum(smax, b3k)
    xg = jax.nn.relu(a3 + x + smax)

    oh0, oh1, oh2 = _knn3_onehots(xg)
    gf = _dot(oh0 + oh1 + oh2, xg) * (1.0 / 3.0)

    out_ref[:, : ROI_SIZE * OUT_CH] = _dot(xg, wi_ref[...])
    out_ref[:, ROI_SIZE * OUT_CH:] = _dot(gf, wctx_ref[...])


def _tc_table(xt, w1t, b1, d2m, d2c, d2p, b2, w3t, b3, w1at, w1bt, bs1,
              ds2, bs2, ws3t, bs3, wi, wctx):
    return pl.pallas_call(
        _tc_body,
        out_shape=jax.ShapeDtypeStruct((TSCALE, NSLOT * OUT_CH), jnp.float32),
    )(xt, w1t, b1, d2m, d2c, d2p, b2, w3t, b3, w1at, w1bt, bs1,
      ds2, bs2, ws3t, bs3, wi, wctx)


_NC = 2
_NS = 16
_NW = _NC * _NS
_APW = NANCH // _NW
_LANES = 16
_NCH = OUT_CH // _LANES


def _sc_body(table_hbm, idx_hbm, wgtb_hbm, bias_hbm, out_hbm,
             idx_v, wsp_v, rows_v, out_v, bias_v, sem):
    wid = lax.axis_index("s") * _NC + lax.axis_index("c")
    base = wid * _APW
    pltpu.sync_copy(idx_hbm.at[pl.ds(base, _APW)], idx_v)
    pltpu.sync_copy(bias_hbm, bias_v)

    def anchor(a, carry):
        pltpu.async_copy(table_hbm.at[idx_v.at[a]], rows_v, sem).wait()
        pltpu.sync_copy(wgtb_hbm.at[pl.ds((base + a) * NTAP, NTAP)], wsp_v)
        acc = [bias_v[pl.ds(c * _LANES, _LANES)] for c in range(_NCH)]
        for j in range(NTAP):
            w = wsp_v[j]
            for c in range(_NCH):
                acc[c] = acc[c] + w * rows_v[j, pl.ds(c * _LANES, _LANES)]
        zero = jnp.zeros((_LANES,), jnp.float32)
        for c in range(_NCH):
            out_v[a, pl.ds(c * _LANES, _LANES)] = jnp.maximum(acc[c], zero)
        return carry

    lax.fori_loop(0, _APW, anchor, 0)
    pltpu.sync_copy(out_v, out_hbm.at[pl.ds(base, _APW)])


@functools.cache
def _sc_align_fn():
    return pl.kernel(
        _sc_body,
        out_type=jax.ShapeDtypeStruct((NANCH, OUT_CH), jnp.float32),
        mesh=plsc.VectorSubcoreMesh(core_axis_name="c", subcore_axis_name="s",
                                    num_cores=_NC, num_subcores=_NS),
        scratch_types=[
            pltpu.VMEM((_APW, NTAP), jnp.int32),
            pltpu.VMEM((NTAP, _LANES), jnp.float32),
            pltpu.VMEM((NTAP, OUT_CH), jnp.float32),
            pltpu.VMEM((_APW, OUT_CH), jnp.float32),
            pltpu.VMEM((OUT_CH,), jnp.float32),
            pltpu.SemaphoreType.DMA,
        ],
    )


def kernel(x, t1_w, t1_b, t2_w, t2_b, t3_w, t3_b, s1_w, s1_b,
           s2_w, s2_b, s3_w, s3_b, c_w, c_b):
    xt = x[0].T

    w1t = t1_w[:, :, 0].T
    d2m = _block_diag_dense(t2_w[:, :, 0])
    d2c = _block_diag_dense(t2_w[:, :, 1])
    d2p = _block_diag_dense(t2_w[:, :, 2])
    w3t = t3_w[:, :, 0].T
    w1at = s1_w[:, :IN_CH, 0, 0].T
    w1bt = s1_w[:, IN_CH:, 0, 0].T
    ds2 = _block_diag_dense(s2_w[:, :, 0, 0])
    ws3t = s3_w[:, :, 0, 0].T

    wc = c_w[:, :, 0, 0].reshape(OUT_CH, IN_CH, NSLOT)
    wi = wc[:, :, :ROI_SIZE].transpose(1, 2, 0).reshape(IN_CH, ROI_SIZE * OUT_CH)
    wctx = wc[:, :, ROI_SIZE:].transpose(1, 2, 0).reshape(IN_CH, CTX_SIZE * OUT_CH)

    table = _tc_table(
        xt, w1t, t1_b[None, :], d2m, d2c, d2p, t2_b[None, :], w3t,
        t3_b[None, :], w1at, w1bt, s1_b[None, :], ds2, s2_b[None, :],
        ws3t, s3_b[None, :], wi, wctx,
    ).reshape(TSCALE * NSLOT, OUT_CH)

    out = _sc_align_fn()(table, jnp.asarray(_IDX_NP), jnp.asarray(_WGTB_NP), c_b)
    return out.reshape(DSCALE, TSCALE, OUT_CH).transpose(2, 0, 1)[None]


# speedup vs baseline: 11.8373x; 1.5570x over previous
import functools

import numpy as np
import jax
import jax.numpy as jnp
from jax import lax
from jax.experimental import pallas as pl
from jax.experimental.pallas import tpu as pltpu
from jax.experimental.pallas import tpu_sc as plsc

IN_CH = 256
OUT_CH = 128
TSCALE = 128
DSCALE = 32
KNN_K = 3
ROI_SIZE = 32
CTX_SIZE = 4
PROP_EXT = 0.5
GROUPS = 32
WIDTH = 4 * GROUPS
NSLOT = ROI_SIZE + CTX_SIZE
NANCH = DSCALE * TSCALE
NTAP = 2 * NSLOT


def _align_consts():
    anchors = []
    for dur_idx in range(DSCALE):
        for start_idx in range(TSCALE):
            end_idx = start_idx + dur_idx + 1
            if end_idx <= TSCALE:
                cl = float(dur_idx + 1)
                anchors.append([start_idx - cl * PROP_EXT, end_idx + cl * PROP_EXT])
            else:
                anchors.append([0.0, 0.0])
    anchors = np.asarray(anchors, dtype=np.float32)

    idx_list, wgt_list = [], []
    for out_size, slot0 in ((ROI_SIZE, 0), (CTX_SIZE, ROI_SIZE)):
        start, end = anchors[:, 0], anchors[:, 1]
        length = np.maximum(end - start, 1.0)
        bin_size = length / float(out_size)
        i = np.arange(out_size, dtype=np.float32)
        pos = start[:, None] + bin_size[:, None] * (i[None, :] + 0.5)
        valid = (pos >= -1.0) & (pos <= float(TSCALE))
        p = np.clip(pos, 0.0, float(TSCALE - 1))
        lo = np.floor(p).astype(np.int32)
        hi = np.minimum(lo + 1, TSCALE - 1)
        w_hi = (p - lo.astype(np.float32)) * valid
        w_lo = (1.0 - (p - lo.astype(np.float32))) * valid
        slot = slot0 + np.arange(out_size, dtype=np.int32)[None, :]
        idx_list += [lo * NSLOT + slot, hi * NSLOT + slot]
        wgt_list += [w_lo, w_hi]
    idx = np.concatenate(idx_list, axis=1).astype(np.int32)
    wgt = np.concatenate(wgt_list, axis=1).astype(np.float32)
    return idx, wgt


_IDX_NP, _WGT_NP = _align_consts()
_WGTB_NP = np.repeat(_WGT_NP.reshape(-1, 1), 16, axis=1).reshape(-1)


def _block_diag_dense(w):
    blocks = w.reshape(GROUPS, 4, 4).transpose(0, 2, 1)
    dense = jnp.zeros((GROUPS, 4, GROUPS, 4), jnp.float32)
    dense = dense.at[jnp.arange(GROUPS), :, jnp.arange(GROUPS), :].set(blocks)
    return dense.reshape(WIDTH, WIDTH)


def _dot(a, b):
    return jnp.dot(a, b, preferred_element_type=jnp.float32)


def _knn3_onehots(z):
    inner = _dot(z, z.T)
    xx = jnp.sum(z * z, axis=1)
    pd = 2.0 * inner - xx[None, :] - xx[:, None]
    iota = lax.broadcasted_iota(jnp.int32, (TSCALE, TSCALE), 1)
    ohs = []
    for _ in range(KNN_K):
        m = jnp.max(pd, axis=1, keepdims=True)
        idx = jnp.min(jnp.where(pd == m, iota, TSCALE), axis=1, keepdims=True)
        sel = iota == idx
        ohs.append(sel.astype(jnp.float32))
        pd = jnp.where(sel, -jnp.inf, pd)
    return ohs


def _tc_body(xt_ref, w1t_ref, b1_ref, d2m_ref, d2c_ref, d2p_ref, b2_ref,
             w3t_ref, b3_ref, w1at_ref, w1bt_ref, bs1_ref, ds2_ref, bs2_ref,
             ws3t_ref, bs3_ref, wi_ref, wctx_ref, out_ref):
    x = xt_ref[...]

    a1 = jax.nn.relu(_dot(x, w1t_ref[...]) + b1_ref[...])
    zrow = jnp.zeros((1, WIDTH), jnp.float32)
    a1m = jnp.concatenate([zrow, a1[:-1, :]], axis=0)
    a1p = jnp.concatenate([a1[1:, :], zrow], axis=0)
    a2 = jax.nn.relu(_dot(a1m, d2m_ref[...]) + _dot(a1, d2c_ref[...])
                     + _dot(a1p, d2p_ref[...]) + b2_ref[...])
    a3 = _dot(a2, w3t_ref[...]) + b3_ref[...]

    smax = None
    for oh in _knn3_onehots(x):
        fk = _dot(oh, x)
        b1k = jax.nn.relu(_dot(fk, w1at_ref[...]) + _dot(x, w1bt_ref[...])
                          + bs1_ref[...])
        b2k = jax.nn.relu(_dot(b1k, ds2_ref[...]) + bs2_ref[...])
        b3k = _dot(b2k, ws3t_ref[...]) + bs3_ref[...]
        smax = b3k if smax is None else jnp.maximum(smax, b3k)
    xg = jax.nn.relu(a3 + x + smax)

    oh0, oh1, oh2 = _knn3_onehots(xg)
    gf = _dot(oh0 + oh1 + oh2, xg) * (1.0 / 3.0)

    out_ref[:, : ROI_SIZE * OUT_CH] = _dot(xg, wi_ref[...])
    out_ref[:, ROI_SIZE * OUT_CH:] = _dot(gf, wctx_ref[...])


def _tc_table(xt, w1t, b1, d2m, d2c, d2p, b2, w3t, b3, w1at, w1bt, bs1,
              ds2, bs2, ws3t, bs3, wi, wctx):
    return pl.pallas_call(
        _tc_body,
        out_shape=jax.ShapeDtypeStruct((TSCALE, NSLOT * OUT_CH), jnp.float32),
    )(xt, w1t, b1, d2m, d2c, d2p, b2, w3t, b3, w1at, w1bt, bs1,
      ds2, bs2, ws3t, bs3, wi, wctx)


_NC = 2
_NS = 16
_NW = _NC * _NS
_APW = NANCH // _NW
_LANES = 16
_NCH = OUT_CH // _LANES


_NBUF = 4
_WCHUNK = NTAP * _LANES


def _sc_body(table_hbm, idx_hbm, wgtb_hbm, bias_hbm, out_hbm,
             idx_v, wsp_v, rows_v, out_v, bias_v, rsems, wsems):
    wid = lax.axis_index("s") * _NC + lax.axis_index("c")
    base = wid * _APW
    pltpu.sync_copy(idx_hbm.at[pl.ds(base, _APW)], idx_v)
    pltpu.sync_copy(bias_hbm, bias_v)

    def start(a, b):
        pltpu.async_copy(table_hbm.at[idx_v.at[a]], rows_v.at[b], rsems[b])
        pltpu.async_copy(wgtb_hbm.at[pl.ds((base + a) * _WCHUNK, _WCHUNK)],
                         wsp_v.at[b], wsems[b])

    for b in range(_NBUF):
        start(b, b)

    def outer(it, carry):
        for b in range(_NBUF):
            a = it * _NBUF + b
            pltpu.make_async_copy(table_hbm.at[idx_v.at[a]], rows_v.at[b],
                                  rsems[b]).wait()
            pltpu.make_async_copy(wgtb_hbm.at[pl.ds((base + a) * _WCHUNK, _WCHUNK)],
                                  wsp_v.at[b], wsems[b]).wait()
            acc = [bias_v[pl.ds(c * _LANES, _LANES)] for c in range(_NCH)]
            for j in range(NTAP):
                w = wsp_v[b, pl.ds(j * _LANES, _LANES)]
                for c in range(_NCH):
                    acc[c] = acc[c] + w * rows_v[b, j, pl.ds(c * _LANES, _LANES)]
            zero = jnp.zeros((_LANES,), jnp.float32)
            for c in range(_NCH):
                out_v[a, pl.ds(c * _LANES, _LANES)] = jnp.maximum(acc[c], zero)

            nxt = a + _NBUF
            @pl.when(nxt < _APW)
            def _():
                start(nxt, b)
        return carry

    lax.fori_loop(0, _APW // _NBUF, outer, 0)
    pltpu.sync_copy(out_v, out_hbm.at[pl.ds(base, _APW)])


@functools.cache
def _sc_align_fn():
    return pl.kernel(
        _sc_body,
        out_type=jax.ShapeDtypeStruct((NANCH, OUT_CH), jnp.float32),
        mesh=plsc.VectorSubcoreMesh(core_axis_name="c", subcore_axis_name="s",
                                    num_cores=_NC, num_subcores=_NS),
        scratch_types=[
            pltpu.VMEM((_APW, NTAP), jnp.int32),
            pltpu.VMEM((_NBUF, _WCHUNK), jnp.float32),
            pltpu.VMEM((_NBUF, NTAP, OUT_CH), jnp.float32),
            pltpu.VMEM((_APW, OUT_CH), jnp.float32),
            pltpu.VMEM((OUT_CH,), jnp.float32),
            [pltpu.SemaphoreType.DMA] * _NBUF,
            [pltpu.SemaphoreType.DMA] * _NBUF,
        ],
    )


def kernel(x, t1_w, t1_b, t2_w, t2_b, t3_w, t3_b, s1_w, s1_b,
           s2_w, s2_b, s3_w, s3_b, c_w, c_b):
    xt = x[0].T

    w1t = t1_w[:, :, 0].T
    d2m = _block_diag_dense(t2_w[:, :, 0])
    d2c = _block_diag_dense(t2_w[:, :, 1])
    d2p = _block_diag_dense(t2_w[:, :, 2])
    w3t = t3_w[:, :, 0].T
    w1at = s1_w[:, :IN_CH, 0, 0].T
    w1bt = s1_w[:, IN_CH:, 0, 0].T
    ds2 = _block_diag_dense(s2_w[:, :, 0, 0])
    ws3t = s3_w[:, :, 0, 0].T

    wc = c_w[:, :, 0, 0].reshape(OUT_CH, IN_CH, NSLOT)
    wi = wc[:, :, :ROI_SIZE].transpose(1, 2, 0).reshape(IN_CH, ROI_SIZE * OUT_CH)
    wctx = wc[:, :, ROI_SIZE:].transpose(1, 2, 0).reshape(IN_CH, CTX_SIZE * OUT_CH)

    table = _tc_table(
        xt, w1t, t1_b[None, :], d2m, d2c, d2p, t2_b[None, :], w3t,
        t3_b[None, :], w1at, w1bt, s1_b[None, :], ds2, s2_b[None, :],
        ws3t, s3_b[None, :], wi, wctx,
    ).reshape(TSCALE * NSLOT, OUT_CH)

    out = _sc_align_fn()(table, jnp.asarray(_IDX_NP), jnp.asarray(_WGTB_NP), c_b)
    return out.reshape(DSCALE, TSCALE, OUT_CH).transpose(2, 0, 1)[None]


# speedup vs baseline: 16.9815x; 1.4346x over previous
import functools

import numpy as np
import jax
import jax.numpy as jnp
from jax import lax
from jax.experimental import pallas as pl
from jax.experimental.pallas import tpu as pltpu
from jax.experimental.pallas import tpu_sc as plsc

IN_CH = 256
OUT_CH = 128
TSCALE = 128
DSCALE = 32
KNN_K = 3
ROI_SIZE = 32
CTX_SIZE = 4
PROP_EXT = 0.5
GROUPS = 32
WIDTH = 4 * GROUPS
NSLOT = ROI_SIZE + CTX_SIZE
NANCH = DSCALE * TSCALE
NTAP = 2 * NSLOT


def _align_consts():
    anchors = []
    for dur_idx in range(DSCALE):
        for start_idx in range(TSCALE):
            end_idx = start_idx + dur_idx + 1
            if end_idx <= TSCALE:
                cl = float(dur_idx + 1)
                anchors.append([start_idx - cl * PROP_EXT, end_idx + cl * PROP_EXT])
            else:
                anchors.append([0.0, 0.0])
    anchors = np.asarray(anchors, dtype=np.float32)

    idx_list, wgt_list = [], []
    for out_size, slot0 in ((ROI_SIZE, 0), (CTX_SIZE, ROI_SIZE)):
        start, end = anchors[:, 0], anchors[:, 1]
        length = np.maximum(end - start, 1.0)
        bin_size = length / float(out_size)
        i = np.arange(out_size, dtype=np.float32)
        pos = start[:, None] + bin_size[:, None] * (i[None, :] + 0.5)
        valid = (pos >= -1.0) & (pos <= float(TSCALE))
        p = np.clip(pos, 0.0, float(TSCALE - 1))
        lo = np.floor(p).astype(np.int32)
        hi = np.minimum(lo + 1, TSCALE - 1)
        w_hi = (p - lo.astype(np.float32)) * valid
        w_lo = (1.0 - (p - lo.astype(np.float32))) * valid
        slot = slot0 + np.arange(out_size, dtype=np.int32)[None, :]
        idx_list += [lo * NSLOT + slot, hi * NSLOT + slot]
        wgt_list += [w_lo, w_hi]
    idx = np.concatenate(idx_list, axis=1).astype(np.int32)
    wgt = np.concatenate(wgt_list, axis=1).astype(np.float32)
    return idx, wgt


_IDX_NP, _WGT_NP = _align_consts()
_WGTB_NP = np.repeat(_WGT_NP.reshape(-1, 1), 16, axis=1).reshape(-1)


def _block_diag_dense(w):
    blocks = w.reshape(GROUPS, 4, 4).transpose(0, 2, 1)
    dense = jnp.zeros((GROUPS, 4, GROUPS, 4), jnp.float32)
    dense = dense.at[jnp.arange(GROUPS), :, jnp.arange(GROUPS), :].set(blocks)
    return dense.reshape(WIDTH, WIDTH)


def _dot(a, b):
    return jnp.dot(a, b, preferred_element_type=jnp.float32)


def _knn3_onehots(z):
    inner = _dot(z, z.T)
    xx = jnp.sum(z * z, axis=1)
    pd = 2.0 * inner - xx[None, :] - xx[:, None]
    iota = lax.broadcasted_iota(jnp.int32, (TSCALE, TSCALE), 1)
    ohs = []
    for _ in range(KNN_K):
        m = jnp.max(pd, axis=1, keepdims=True)
        idx = jnp.min(jnp.where(pd == m, iota, TSCALE), axis=1, keepdims=True)
        sel = iota == idx
        ohs.append(sel.astype(jnp.float32))
        pd = jnp.where(sel, -jnp.inf, pd)
    return ohs


def _tc_body(xt_ref, w1t_ref, b1_ref, d2m_ref, d2c_ref, d2p_ref, b2_ref,
             w3t_ref, b3_ref, w1at_ref, w1bt_ref, bs1_ref, ds2_ref, bs2_ref,
             ws3t_ref, bs3_ref, wi_ref, wctx_ref, out_ref):
    x = xt_ref[...]

    a1 = jax.nn.relu(_dot(x, w1t_ref[...]) + b1_ref[...])
    zrow = jnp.zeros((1, WIDTH), jnp.float32)
    a1m = jnp.concatenate([zrow, a1[:-1, :]], axis=0)
    a1p = jnp.concatenate([a1[1:, :], zrow], axis=0)
    a2 = jax.nn.relu(_dot(a1m, d2m_ref[...]) + _dot(a1, d2c_ref[...])
                     + _dot(a1p, d2p_ref[...]) + b2_ref[...])
    a3 = _dot(a2, w3t_ref[...]) + b3_ref[...]

    smax = None
    for oh in _knn3_onehots(x):
        fk = _dot(oh, x)
        b1k = jax.nn.relu(_dot(fk, w1at_ref[...]) + _dot(x, w1bt_ref[...])
                          + bs1_ref[...])
        b2k = jax.nn.relu(_dot(b1k, ds2_ref[...]) + bs2_ref[...])
        b3k = _dot(b2k, ws3t_ref[...]) + bs3_ref[...]
        smax = b3k if smax is None else jnp.maximum(smax, b3k)
    xg = jax.nn.relu(a3 + x + smax)

    oh0, oh1, oh2 = _knn3_onehots(xg)
    gf = _dot(oh0 + oh1 + oh2, xg) * (1.0 / 3.0)

    out_ref[:, : ROI_SIZE * OUT_CH] = _dot(xg, wi_ref[...])
    out_ref[:, ROI_SIZE * OUT_CH:] = _dot(gf, wctx_ref[...])


def _tc_table(xt, w1t, b1, d2m, d2c, d2p, b2, w3t, b3, w1at, w1bt, bs1,
              ds2, bs2, ws3t, bs3, wi, wctx):
    return pl.pallas_call(
        _tc_body,
        out_shape=jax.ShapeDtypeStruct((TSCALE, NSLOT * OUT_CH), jnp.float32),
    )(xt, w1t, b1, d2m, d2c, d2p, b2, w3t, b3, w1at, w1bt, bs1,
      ds2, bs2, ws3t, bs3, wi, wctx)


_NC = 2
_NS = 16
_NW = _NC * _NS
_APW = NANCH // _NW
_LANES = 16
_NCH = OUT_CH // _LANES


_NBUF = 4
_WCHUNK = NTAP * _LANES


def _sc_body(table_hbm, idx_hbm, wgtb_hbm, bias_hbm, out_hbm,
             idx_v, wsp_v, rows_v, out_v, bias_v, rsems, wsems):
    wid = lax.axis_index("s") * _NC + lax.axis_index("c")
    base = wid * _APW
    pltpu.sync_copy(idx_hbm.at[pl.ds(base, _APW)], idx_v)
    pltpu.sync_copy(bias_hbm, bias_v)

    def start(a, b):
        pltpu.async_copy(table_hbm.at[idx_v.at[a]], rows_v.at[b], rsems[b])
        pltpu.async_copy(wgtb_hbm.at[pl.ds((base + a) * _WCHUNK, _WCHUNK)],
                         wsp_v.at[b], wsems[b])

    for b in range(_NBUF):
        start(b, b)

    def outer(it, carry):
        for b in range(_NBUF):
            a = it * _NBUF + b
            pltpu.make_async_copy(table_hbm.at[idx_v.at[a]], rows_v.at[b],
                                  rsems[b]).wait()
            pltpu.make_async_copy(wgtb_hbm.at[pl.ds((base + a) * _WCHUNK, _WCHUNK)],
                                  wsp_v.at[b], wsems[b]).wait()
            acc0 = tuple(bias_v[pl.ds(c * _LANES, _LANES)] for c in range(_NCH))

            def accum(j, acc):
                w = wsp_v[b, pl.ds(j * _LANES, _LANES)]
                return tuple(
                    acc[c] + w * rows_v[b, j, pl.ds(c * _LANES, _LANES)]
                    for c in range(_NCH))

            acc = plsc.parallel_loop(0, NTAP, unroll=8, carry=acc0)(accum)
            zero = jnp.zeros((_LANES,), jnp.float32)
            for c in range(_NCH):
                out_v[a, pl.ds(c * _LANES, _LANES)] = jnp.maximum(acc[c], zero)

            nxt = a + _NBUF
            @pl.when(nxt < _APW)
            def _():
                start(nxt, b)
        return carry

    lax.fori_loop(0, _APW // _NBUF, outer, 0)
    pltpu.sync_copy(out_v, out_hbm.at[pl.ds(base, _APW)])


@functools.cache
def _sc_align_fn():
    return pl.kernel(
        _sc_body,
        out_type=jax.ShapeDtypeStruct((NANCH, OUT_CH), jnp.float32),
        mesh=plsc.VectorSubcoreMesh(core_axis_name="c", subcore_axis_name="s",
                                    num_cores=_NC, num_subcores=_NS),
        scratch_types=[
            pltpu.VMEM((_APW, NTAP), jnp.int32),
            pltpu.VMEM((_NBUF, _WCHUNK), jnp.float32),
            pltpu.VMEM((_NBUF, NTAP, OUT_CH), jnp.float32),
            pltpu.VMEM((_APW, OUT_CH), jnp.float32),
            pltpu.VMEM((OUT_CH,), jnp.float32),
            [pltpu.SemaphoreType.DMA] * _NBUF,
            [pltpu.SemaphoreType.DMA] * _NBUF,
        ],
    )


def kernel(x, t1_w, t1_b, t2_w, t2_b, t3_w, t3_b, s1_w, s1_b,
           s2_w, s2_b, s3_w, s3_b, c_w, c_b):
    xt = x[0].T

    w1t = t1_w[:, :, 0].T
    d2m = _block_diag_dense(t2_w[:, :, 0])
    d2c = _block_diag_dense(t2_w[:, :, 1])
    d2p = _block_diag_dense(t2_w[:, :, 2])
    w3t = t3_w[:, :, 0].T
    w1at = s1_w[:, :IN_CH, 0, 0].T
    w1bt = s1_w[:, IN_CH:, 0, 0].T
    ds2 = _block_diag_dense(s2_w[:, :, 0, 0])
    ws3t = s3_w[:, :, 0, 0].T

    wc = c_w[:, :, 0, 0].reshape(OUT_CH, IN_CH, NSLOT)
    wi = wc[:, :, :ROI_SIZE].transpose(1, 2, 0).reshape(IN_CH, ROI_SIZE * OUT_CH)
    wctx = wc[:, :, ROI_SIZE:].transpose(1, 2, 0).reshape(IN_CH, CTX_SIZE * OUT_CH)

    table = _tc_table(
        xt, w1t, t1_b[None, :], d2m, d2c, d2p, t2_b[None, :], w3t,
        t3_b[None, :], w1at, w1bt, s1_b[None, :], ds2, s2_b[None, :],
        ws3t, s3_b[None, :], wi, wctx,
    ).reshape(TSCALE * NSLOT, OUT_CH)

    out = _sc_align_fn()(table, jnp.asarray(_IDX_NP), jnp.asarray(_WGTB_NP), c_b)
    return out.reshape(DSCALE, TSCALE, OUT_CH).transpose(2, 0, 1)[None]


# speedup vs baseline: 17.0389x; 1.0034x over previous
import functools

import numpy as np
import jax
import jax.numpy as jnp
from jax import lax
from jax.experimental import pallas as pl
from jax.experimental.pallas import tpu as pltpu
from jax.experimental.pallas import tpu_sc as plsc

IN_CH = 256
OUT_CH = 128
TSCALE = 128
DSCALE = 32
KNN_K = 3
ROI_SIZE = 32
CTX_SIZE = 4
PROP_EXT = 0.5
GROUPS = 32
WIDTH = 4 * GROUPS
NSLOT = ROI_SIZE + CTX_SIZE
NANCH = DSCALE * TSCALE
NTAP = 2 * NSLOT
PAIR = 2 * OUT_CH


def _align_consts():
    anchors = []
    for dur_idx in range(DSCALE):
        for start_idx in range(TSCALE):
            end_idx = start_idx + dur_idx + 1
            if end_idx <= TSCALE:
                cl = float(dur_idx + 1)
                anchors.append([start_idx - cl * PROP_EXT, end_idx + cl * PROP_EXT])
            else:
                anchors.append([0.0, 0.0])
    anchors = np.asarray(anchors, dtype=np.float32)

    idx_list, wgt_list = [], []
    for out_size, slot0 in ((ROI_SIZE, 0), (CTX_SIZE, ROI_SIZE)):
        start, end = anchors[:, 0], anchors[:, 1]
        length = np.maximum(end - start, 1.0)
        bin_size = length / float(out_size)
        i = np.arange(out_size, dtype=np.float32)
        pos = start[:, None] + bin_size[:, None] * (i[None, :] + 0.5)
        valid = (pos >= -1.0) & (pos <= float(TSCALE))
        p = np.clip(pos, 0.0, float(TSCALE - 1))
        lo = np.floor(p).astype(np.int32)
        hi = np.minimum(lo + 1, TSCALE - 1)
        w_hi = (p - lo.astype(np.float32)) * valid
        w_lo = (1.0 - (p - lo.astype(np.float32))) * valid
        slot = slot0 + np.arange(out_size, dtype=np.int32)[None, :]
        idx_list += [lo * NSLOT + slot]
        wgt_list += [np.stack([w_lo, w_hi], axis=2)]
    idx = np.concatenate(idx_list, axis=1).astype(np.int32)
    wgt = np.concatenate(
        [w.reshape(NANCH, -1) for w in wgt_list], axis=1).astype(np.float32)
    return idx, wgt


_IDX_NP, _WGT_NP = _align_consts()
_WGTB_NP = np.repeat(_WGT_NP.reshape(-1, 1), 16, axis=1).reshape(-1)


def _block_diag_dense(w):
    blocks = w.reshape(GROUPS, 4, 4).transpose(0, 2, 1)
    dense = jnp.zeros((GROUPS, 4, GROUPS, 4), jnp.float32)
    dense = dense.at[jnp.arange(GROUPS), :, jnp.arange(GROUPS), :].set(blocks)
    return dense.reshape(WIDTH, WIDTH)


def _dot(a, b):
    return jnp.dot(a, b, preferred_element_type=jnp.float32)


def _knn3_onehots(z):
    inner = _dot(z, z.T)
    xx = jnp.sum(z * z, axis=1)
    pd = 2.0 * inner - xx[None, :] - xx[:, None]
    iota = lax.broadcasted_iota(jnp.int32, (TSCALE, TSCALE), 1)
    ohs = []
    for _ in range(KNN_K):
        m = jnp.max(pd, axis=1, keepdims=True)
        idx = jnp.min(jnp.where(pd == m, iota, TSCALE), axis=1, keepdims=True)
        sel = iota == idx
        ohs.append(sel.astype(jnp.float32))
        pd = jnp.where(sel, -jnp.inf, pd)
    return ohs


def _tc_body(xt_ref, w1t_ref, b1_ref, d2m_ref, d2c_ref, d2p_ref, b2_ref,
             w3t_ref, b3_ref, w1at_ref, w1bt_ref, bs1_ref, ds2_ref, bs2_ref,
             ws3t_ref, bs3_ref, wi_ref, wctx_ref, out_ref):
    x = xt_ref[...]

    a1 = jax.nn.relu(_dot(x, w1t_ref[...]) + b1_ref[...])
    zrow = jnp.zeros((1, WIDTH), jnp.float32)
    a1m = jnp.concatenate([zrow, a1[:-1, :]], axis=0)
    a1p = jnp.concatenate([a1[1:, :], zrow], axis=0)
    a2 = jax.nn.relu(_dot(a1m, d2m_ref[...]) + _dot(a1, d2c_ref[...])
                     + _dot(a1p, d2p_ref[...]) + b2_ref[...])
    a3 = _dot(a2, w3t_ref[...]) + b3_ref[...]

    smax = None
    for oh in _knn3_onehots(x):
        fk = _dot(oh, x)
        b1k = jax.nn.relu(_dot(fk, w1at_ref[...]) + _dot(x, w1bt_ref[...])
                          + bs1_ref[...])
        b2k = jax.nn.relu(_dot(b1k, ds2_ref[...]) + bs2_ref[...])
        b3k = _dot(b2k, ws3t_ref[...]) + bs3_ref[...]
        smax = b3k if smax is None else jnp.maximum(smax, b3k)
    xg = jax.nn.relu(a3 + x + smax)

    oh0, oh1, oh2 = _knn3_onehots(xg)
    gf = _dot(oh0 + oh1 + oh2, xg) * (1.0 / 3.0)

    out_ref[:, : ROI_SIZE * OUT_CH] = _dot(xg, wi_ref[...])
    out_ref[:, ROI_SIZE * OUT_CH:] = _dot(gf, wctx_ref[...])


def _tc_table(xt, w1t, b1, d2m, d2c, d2p, b2, w3t, b3, w1at, w1bt, bs1,
              ds2, bs2, ws3t, bs3, wi, wctx):
    return pl.pallas_call(
        _tc_body,
        out_shape=jax.ShapeDtypeStruct((TSCALE, NSLOT * OUT_CH), jnp.float32),
    )(xt, w1t, b1, d2m, d2c, d2p, b2, w3t, b3, w1at, w1bt, bs1,
      ds2, bs2, ws3t, bs3, wi, wctx)


_NC = 2
_NS = 16
_NW = _NC * _NS
_APW = NANCH // _NW
_LANES = 16
_NCH = OUT_CH // _LANES


_NBUF = 4
_WCHUNK = NTAP * _LANES


def _sc_body(table_hbm, idx_hbm, wgtb_hbm, bias_hbm, out_hbm,
             idx_v, wsp_v, rows_v, out_v, bias_v, rsems, wsems):
    wid = lax.axis_index("s") * _NC + lax.axis_index("c")
    base = wid * _APW
    pltpu.sync_copy(idx_hbm.at[pl.ds(base, _APW)], idx_v)
    pltpu.sync_copy(bias_hbm, bias_v)

    def start(a, b):
        pltpu.async_copy(table_hbm.at[idx_v.at[a]], rows_v.at[b], rsems[b])
        pltpu.async_copy(wgtb_hbm.at[pl.ds((base + a) * _WCHUNK, _WCHUNK)],
                         wsp_v.at[b], wsems[b])

    for b in range(_NBUF):
        start(b, b)

    def outer(it, carry):
        for b in range(_NBUF):
            a = it * _NBUF + b
            pltpu.make_async_copy(table_hbm.at[idx_v.at[a]], rows_v.at[b],
                                  rsems[b]).wait()
            pltpu.make_async_copy(wgtb_hbm.at[pl.ds((base + a) * _WCHUNK, _WCHUNK)],
                                  wsp_v.at[b], wsems[b]).wait()
            acc0 = tuple(bias_v[pl.ds(c * _LANES, _LANES)] for c in range(_NCH))

            def accum(i, acc):
                wlo = wsp_v[b, pl.ds((2 * i) * _LANES, _LANES)]
                whi = wsp_v[b, pl.ds((2 * i + 1) * _LANES, _LANES)]
                return tuple(
                    acc[c] + wlo * rows_v[b, i, pl.ds(c * _LANES, _LANES)]
                    + whi * rows_v[b, i, pl.ds(OUT_CH + c * _LANES, _LANES)]
                    for c in range(_NCH))

            acc = plsc.parallel_loop(0, NSLOT, unroll=6, carry=acc0)(accum)
            zero = jnp.zeros((_LANES,), jnp.float32)
            for c in range(_NCH):
                out_v[a, pl.ds(c * _LANES, _LANES)] = jnp.maximum(acc[c], zero)

            nxt = a + _NBUF
            @pl.when(nxt < _APW)
            def _():
                start(nxt, b)
        return carry

    lax.fori_loop(0, _APW // _NBUF, outer, 0)
    pltpu.sync_copy(out_v, out_hbm.at[pl.ds(base, _APW)])


@functools.cache
def _sc_align_fn():
    return pl.kernel(
        _sc_body,
        out_type=jax.ShapeDtypeStruct((NANCH, OUT_CH), jnp.float32),
        mesh=plsc.VectorSubcoreMesh(core_axis_name="c", subcore_axis_name="s",
                                    num_cores=_NC, num_subcores=_NS),
        scratch_types=[
            pltpu.VMEM((_APW, NSLOT), jnp.int32),
            pltpu.VMEM((_NBUF, _WCHUNK), jnp.float32),
            pltpu.VMEM((_NBUF, NSLOT, PAIR), jnp.float32),
            pltpu.VMEM((_APW, OUT_CH), jnp.float32),
            pltpu.VMEM((OUT_CH,), jnp.float32),
            [pltpu.SemaphoreType.DMA] * _NBUF,
            [pltpu.SemaphoreType.DMA] * _NBUF,
        ],
    )


def kernel(x, t1_w, t1_b, t2_w, t2_b, t3_w, t3_b, s1_w, s1_b,
           s2_w, s2_b, s3_w, s3_b, c_w, c_b):
    xt = x[0].T

    w1t = t1_w[:, :, 0].T
    d2m = _block_diag_dense(t2_w[:, :, 0])
    d2c = _block_diag_dense(t2_w[:, :, 1])
    d2p = _block_diag_dense(t2_w[:, :, 2])
    w3t = t3_w[:, :, 0].T
    w1at = s1_w[:, :IN_CH, 0, 0].T
    w1bt = s1_w[:, IN_CH:, 0, 0].T
    ds2 = _block_diag_dense(s2_w[:, :, 0, 0])
    ws3t = s3_w[:, :, 0, 0].T

    wc = c_w[:, :, 0, 0].reshape(OUT_CH, IN_CH, NSLOT)
    wi = wc[:, :, :ROI_SIZE].transpose(1, 2, 0).reshape(IN_CH, ROI_SIZE * OUT_CH)
    wctx = wc[:, :, ROI_SIZE:].transpose(1, 2, 0).reshape(IN_CH, CTX_SIZE * OUT_CH)

    table = _tc_table(
        xt, w1t, t1_b[None, :], d2m, d2c, d2p, t2_b[None, :], w3t,
        t3_b[None, :], w1at, w1bt, s1_b[None, :], ds2, s2_b[None, :],
        ws3t, s3_b[None, :], wi, wctx,
    ).reshape(TSCALE, NSLOT, OUT_CH)

    tnext = jnp.concatenate([table[1:], table[-1:]], axis=0)
    ptab = jnp.concatenate([table, tnext], axis=2).reshape(TSCALE * NSLOT, PAIR)

    out = _sc_align_fn()(ptab, jnp.asarray(_IDX_NP), jnp.asarray(_WGTB_NP), c_b)
    return out.reshape(DSCALE, TSCALE, OUT_CH).transpose(2, 0, 1)[None]


# speedup vs baseline: 25.0342x; 1.4692x over previous
import functools

import numpy as np
import jax
import jax.numpy as jnp
from jax import lax
from jax.experimental import pallas as pl
from jax.experimental.pallas import tpu as pltpu
from jax.experimental.pallas import tpu_sc as plsc

IN_CH = 256
OUT_CH = 128
TSCALE = 128
DSCALE = 32
KNN_K = 3
ROI_SIZE = 32
CTX_SIZE = 4
PROP_EXT = 0.5
GROUPS = 32
WIDTH = 4 * GROUPS
NSLOT = ROI_SIZE + CTX_SIZE
NANCH = DSCALE * TSCALE
NTAP = 2 * NSLOT
PAIR = 2 * OUT_CH


def _align_consts():
    anchors = []
    for dur_idx in range(DSCALE):
        for start_idx in range(TSCALE):
            end_idx = start_idx + dur_idx + 1
            if end_idx <= TSCALE:
                cl = float(dur_idx + 1)
                anchors.append([start_idx - cl * PROP_EXT, end_idx + cl * PROP_EXT])
            else:
                anchors.append([0.0, 0.0])
    anchors = np.asarray(anchors, dtype=np.float32)

    idx_list, wgt_list = [], []
    for out_size, slot0 in ((ROI_SIZE, 0), (CTX_SIZE, ROI_SIZE)):
        start, end = anchors[:, 0], anchors[:, 1]
        length = np.maximum(end - start, 1.0)
        bin_size = length / float(out_size)
        i = np.arange(out_size, dtype=np.float32)
        pos = start[:, None] + bin_size[:, None] * (i[None, :] + 0.5)
        valid = (pos >= -1.0) & (pos <= float(TSCALE))
        p = np.clip(pos, 0.0, float(TSCALE - 1))
        lo = np.floor(p).astype(np.int32)
        hi = np.minimum(lo + 1, TSCALE - 1)
        w_hi = (p - lo.astype(np.float32)) * valid
        w_lo = (1.0 - (p - lo.astype(np.float32))) * valid
        slot = slot0 + np.arange(out_size, dtype=np.int32)[None, :]
        idx_list += [lo * NSLOT + slot, hi * NSLOT + slot]
        wgt_list += [w_lo, w_hi]
    idx = np.concatenate(idx_list, axis=1).astype(np.int32)
    wgt = np.concatenate(wgt_list, axis=1).astype(np.float32)
    return idx, wgt


_IDX_NP, _WGT_NP = _align_consts()
_WGTB_NP = np.repeat(_WGT_NP.reshape(-1, 1), 16, axis=1).reshape(-1)


def _block_diag_dense(w):
    blocks = w.reshape(GROUPS, 4, 4).transpose(0, 2, 1)
    dense = jnp.zeros((GROUPS, 4, GROUPS, 4), jnp.float32)
    dense = dense.at[jnp.arange(GROUPS), :, jnp.arange(GROUPS), :].set(blocks)
    return dense.reshape(WIDTH, WIDTH)


def _dot(a, b):
    return jnp.dot(a, b, preferred_element_type=jnp.float32)


def _knn3_onehots(z):
    inner = _dot(z, z.T)
    xx = jnp.sum(z * z, axis=1)
    pd = 2.0 * inner - xx[None, :] - xx[:, None]
    iota = lax.broadcasted_iota(jnp.int32, (TSCALE, TSCALE), 1)
    ohs = []
    for _ in range(KNN_K):
        m = jnp.max(pd, axis=1, keepdims=True)
        idx = jnp.min(jnp.where(pd == m, iota, TSCALE), axis=1, keepdims=True)
        sel = iota == idx
        ohs.append(sel.astype(jnp.float32))
        pd = jnp.where(sel, -jnp.inf, pd)
    return ohs


def _tc_body(xt_ref, w1t_ref, b1_ref, d2m_ref, d2c_ref, d2p_ref, b2_ref,
             w3t_ref, b3_ref, w1at_ref, w1bt_ref, bs1_ref, ds2_ref, bs2_ref,
             ws3t_ref, bs3_ref, wi_ref, wctx_ref, out_ref):
    x = xt_ref[...]

    a1 = jax.nn.relu(_dot(x, w1t_ref[...]) + b1_ref[...])
    zrow = jnp.zeros((1, WIDTH), jnp.float32)
    a1m = jnp.concatenate([zrow, a1[:-1, :]], axis=0)
    a1p = jnp.concatenate([a1[1:, :], zrow], axis=0)
    a2 = jax.nn.relu(_dot(a1m, d2m_ref[...]) + _dot(a1, d2c_ref[...])
                     + _dot(a1p, d2p_ref[...]) + b2_ref[...])
    a3 = _dot(a2, w3t_ref[...]) + b3_ref[...]

    smax = None
    for oh in _knn3_onehots(x):
        fk = _dot(oh, x)
        b1k = jax.nn.relu(_dot(fk, w1at_ref[...]) + _dot(x, w1bt_ref[...])
                          + bs1_ref[...])
        b2k = jax.nn.relu(_dot(b1k, ds2_ref[...]) + bs2_ref[...])
        b3k = _dot(b2k, ws3t_ref[...]) + bs3_ref[...]
        smax = b3k if smax is None else jnp.maximum(smax, b3k)
    xg = jax.nn.relu(a3 + x + smax)

    oh0, oh1, oh2 = _knn3_onehots(xg)
    gf = _dot(oh0 + oh1 + oh2, xg) * (1.0 / 3.0)

    out_ref[:, : ROI_SIZE * OUT_CH] = _dot(xg, wi_ref[...])
    out_ref[:, ROI_SIZE * OUT_CH:] = _dot(gf, wctx_ref[...])


def _tc_table(xt, w1t, b1, d2m, d2c, d2p, b2, w3t, b3, w1at, w1bt, bs1,
              ds2, bs2, ws3t, bs3, wi, wctx):
    return pl.pallas_call(
        _tc_body,
        out_shape=jax.ShapeDtypeStruct((TSCALE, NSLOT * OUT_CH), jnp.float32),
    )(xt, w1t, b1, d2m, d2c, d2p, b2, w3t, b3, w1at, w1bt, bs1,
      ds2, bs2, ws3t, bs3, wi, wctx)


_NC = 2
_NS = 16
_NW = _NC * _NS
_APW = NANCH // _NW
_LANES = 16
_NCH = OUT_CH // _LANES


_NBUF = 4
_WCHUNK = NTAP * _LANES


def _sc_body(table_hbm, idx_hbm, wgtb_hbm, bias_hbm, out_hbm,
             idx_v, wsp_v, rows_v, out_v, bias_v, tab_sh, rsems, wsems):
    wid = lax.axis_index("s") * _NC + lax.axis_index("c")
    base = wid * _APW
    pltpu.sync_copy(idx_hbm.at[pl.ds(base, _APW)], idx_v)
    pltpu.sync_copy(bias_hbm, bias_v)

    @pl.when(lax.axis_index("s") == 0)
    def _():
        pltpu.sync_copy(table_hbm, tab_sh)
    plsc.subcore_barrier()

    def start(a, b):
        pltpu.async_copy(tab_sh.at[idx_v.at[a]], rows_v.at[b], rsems[b])
        pltpu.async_copy(wgtb_hbm.at[pl.ds((base + a) * _WCHUNK, _WCHUNK)],
                         wsp_v.at[b], wsems[b])

    for b in range(_NBUF):
        start(b, b)

    def outer(it, carry):
        for b in range(_NBUF):
            a = it * _NBUF + b
            pltpu.make_async_copy(table_hbm.at[idx_v.at[a]], rows_v.at[b],
                                  rsems[b]).wait()
            pltpu.make_async_copy(wgtb_hbm.at[pl.ds((base + a) * _WCHUNK, _WCHUNK)],
                                  wsp_v.at[b], wsems[b]).wait()
            acc0 = tuple(bias_v[pl.ds(c * _LANES, _LANES)] for c in range(_NCH))

            def accum(j, acc):
                w = wsp_v[b, pl.ds(j * _LANES, _LANES)]
                return tuple(
                    acc[c] + w * rows_v[b, j, pl.ds(c * _LANES, _LANES)]
                    for c in range(_NCH))

            acc = plsc.parallel_loop(0, NTAP, unroll=8, carry=acc0)(accum)
            zero = jnp.zeros((_LANES,), jnp.float32)
            for c in range(_NCH):
                out_v[a, pl.ds(c * _LANES, _LANES)] = jnp.maximum(acc[c], zero)

            nxt = a + _NBUF
            @pl.when(nxt < _APW)
            def _():
                start(nxt, b)
        return carry

    lax.fori_loop(0, _APW // _NBUF, outer, 0)
    pltpu.sync_copy(out_v, out_hbm.at[pl.ds(base, _APW)])


@functools.cache
def _sc_align_fn():
    return pl.kernel(
        _sc_body,
        out_type=jax.ShapeDtypeStruct((NANCH, OUT_CH), jnp.float32),
        mesh=plsc.VectorSubcoreMesh(core_axis_name="c", subcore_axis_name="s",
                                    num_cores=_NC, num_subcores=_NS),
        scratch_types=[
            pltpu.VMEM((_APW, NTAP), jnp.int32),
            pltpu.VMEM((_NBUF, _WCHUNK), jnp.float32),
            pltpu.VMEM((_NBUF, NTAP, OUT_CH), jnp.float32),
            pltpu.VMEM((_APW, OUT_CH), jnp.float32),
            pltpu.VMEM((OUT_CH,), jnp.float32),
            pltpu.VMEM_SHARED((TSCALE * NSLOT, OUT_CH), jnp.float32),
            [pltpu.SemaphoreType.DMA] * _NBUF,
            [pltpu.SemaphoreType.DMA] * _NBUF,
        ],
    )


def kernel(x, t1_w, t1_b, t2_w, t2_b, t3_w, t3_b, s1_w, s1_b,
           s2_w, s2_b, s3_w, s3_b, c_w, c_b):
    xt = x[0].T

    w1t = t1_w[:, :, 0].T
    d2m = _block_diag_dense(t2_w[:, :, 0])
    d2c = _block_diag_dense(t2_w[:, :, 1])
    d2p = _block_diag_dense(t2_w[:, :, 2])
    w3t = t3_w[:, :, 0].T
    w1at = s1_w[:, :IN_CH, 0, 0].T
    w1bt = s1_w[:, IN_CH:, 0, 0].T
    ds2 = _block_diag_dense(s2_w[:, :, 0, 0])
    ws3t = s3_w[:, :, 0, 0].T

    wc = c_w[:, :, 0, 0].reshape(OUT_CH, IN_CH, NSLOT)
    wi = wc[:, :, :ROI_SIZE].transpose(1, 2, 0).reshape(IN_CH, ROI_SIZE * OUT_CH)
    wctx = wc[:, :, ROI_SIZE:].transpose(1, 2, 0).reshape(IN_CH, CTX_SIZE * OUT_CH)

    table = _tc_table(
        xt, w1t, t1_b[None, :], d2m, d2c, d2p, t2_b[None, :], w3t,
        t3_b[None, :], w1at, w1bt, s1_b[None, :], ds2, s2_b[None, :],
        ws3t, s3_b[None, :], wi, wctx,
    ).reshape(TSCALE * NSLOT, OUT_CH)

    out = _sc_align_fn()(table, jnp.asarray(_IDX_NP), jnp.asarray(_WGTB_NP), c_b)
    return out.reshape(DSCALE, TSCALE, OUT_CH).transpose(2, 0, 1)[None]


# speedup vs baseline: 25.0506x; 1.0007x over previous
import functools

import numpy as np
import jax
import jax.numpy as jnp
from jax import lax
from jax.experimental import pallas as pl
from jax.experimental.pallas import tpu as pltpu
from jax.experimental.pallas import tpu_sc as plsc

IN_CH = 256
OUT_CH = 128
TSCALE = 128
DSCALE = 32
KNN_K = 3
ROI_SIZE = 32
CTX_SIZE = 4
PROP_EXT = 0.5
GROUPS = 32
WIDTH = 4 * GROUPS
NSLOT = ROI_SIZE + CTX_SIZE
NANCH = DSCALE * TSCALE
NTAP = 2 * NSLOT
PAIR = 2 * OUT_CH


def _align_consts():
    anchors = []
    for dur_idx in range(DSCALE):
        for start_idx in range(TSCALE):
            end_idx = start_idx + dur_idx + 1
            if end_idx <= TSCALE:
                cl = float(dur_idx + 1)
                anchors.append([start_idx - cl * PROP_EXT, end_idx + cl * PROP_EXT])
            else:
                anchors.append([0.0, 0.0])
    anchors = np.asarray(anchors, dtype=np.float32)

    idx_list, wgt_list = [], []
    for out_size, slot0 in ((ROI_SIZE, 0), (CTX_SIZE, ROI_SIZE)):
        start, end = anchors[:, 0], anchors[:, 1]
        length = np.maximum(end - start, 1.0)
        bin_size = length / float(out_size)
        i = np.arange(out_size, dtype=np.float32)
        pos = start[:, None] + bin_size[:, None] * (i[None, :] + 0.5)
        valid = (pos >= -1.0) & (pos <= float(TSCALE))
        p = np.clip(pos, 0.0, float(TSCALE - 1))
        lo = np.floor(p).astype(np.int32)
        hi = np.minimum(lo + 1, TSCALE - 1)
        w_hi = (p - lo.astype(np.float32)) * valid
        w_lo = (1.0 - (p - lo.astype(np.float32))) * valid
        slot = slot0 + np.arange(out_size, dtype=np.int32)[None, :]
        idx_list += [lo * NSLOT + slot, hi * NSLOT + slot]
        wgt_list += [w_lo, w_hi]
    idx = np.concatenate(idx_list, axis=1).astype(np.int32)
    wgt = np.concatenate(wgt_list, axis=1).astype(np.float32)
    return idx, wgt


_IDX_NP, _WGT_NP = _align_consts()
_WGTB_NP = np.repeat(_WGT_NP.reshape(-1, 1), 16, axis=1).reshape(-1)


def _block_diag_dense(w):
    blocks = w.reshape(GROUPS, 4, 4).transpose(0, 2, 1)
    dense = jnp.zeros((GROUPS, 4, GROUPS, 4), jnp.float32)
    dense = dense.at[jnp.arange(GROUPS), :, jnp.arange(GROUPS), :].set(blocks)
    return dense.reshape(WIDTH, WIDTH)


def _dot(a, b):
    return jnp.dot(a, b, preferred_element_type=jnp.float32)


def _knn3_onehots(z):
    inner = _dot(z, z.T)
    xx = jnp.sum(z * z, axis=1)
    pd = 2.0 * inner - xx[None, :] - xx[:, None]
    iota = lax.broadcasted_iota(jnp.int32, (TSCALE, TSCALE), 1)
    ohs = []
    for _ in range(KNN_K):
        m = jnp.max(pd, axis=1, keepdims=True)
        idx = jnp.min(jnp.where(pd == m, iota, TSCALE), axis=1, keepdims=True)
        sel = iota == idx
        ohs.append(sel.astype(jnp.float32))
        pd = jnp.where(sel, -jnp.inf, pd)
    return ohs


def _tc_body(xt_ref, w1t_ref, b1_ref, d2m_ref, d2c_ref, d2p_ref, b2_ref,
             w3t_ref, b3_ref, w1at_ref, w1bt_ref, bs1_ref, ds2_ref, bs2_ref,
             ws3t_ref, bs3_ref, wi_ref, wctx_ref, out_ref):
    x = xt_ref[...]

    a1 = jax.nn.relu(_dot(x, w1t_ref[...]) + b1_ref[...])
    zrow = jnp.zeros((1, WIDTH), jnp.float32)
    a1m = jnp.concatenate([zrow, a1[:-1, :]], axis=0)
    a1p = jnp.concatenate([a1[1:, :], zrow], axis=0)
    a2 = jax.nn.relu(_dot(a1m, d2m_ref[...]) + _dot(a1, d2c_ref[...])
                     + _dot(a1p, d2p_ref[...]) + b2_ref[...])
    a3 = _dot(a2, w3t_ref[...]) + b3_ref[...]

    smax = None
    for oh in _knn3_onehots(x):
        fk = _dot(oh, x)
        b1k = jax.nn.relu(_dot(fk, w1at_ref[...]) + _dot(x, w1bt_ref[...])
                          + bs1_ref[...])
        b2k = jax.nn.relu(_dot(b1k, ds2_ref[...]) + bs2_ref[...])
        b3k = _dot(b2k, ws3t_ref[...]) + bs3_ref[...]
        smax = b3k if smax is None else jnp.maximum(smax, b3k)
    xg = jax.nn.relu(a3 + x + smax)

    oh0, oh1, oh2 = _knn3_onehots(xg)
    gf = _dot(oh0 + oh1 + oh2, xg) * (1.0 / 3.0)

    out_ref[:, : ROI_SIZE * OUT_CH] = _dot(xg, wi_ref[...])
    out_ref[:, ROI_SIZE * OUT_CH:] = _dot(gf, wctx_ref[...])


def _tc_table(xt, w1t, b1, d2m, d2c, d2p, b2, w3t, b3, w1at, w1bt, bs1,
              ds2, bs2, ws3t, bs3, wi, wctx):
    return pl.pallas_call(
        _tc_body,
        out_shape=jax.ShapeDtypeStruct((TSCALE, NSLOT * OUT_CH), jnp.float32),
    )(xt, w1t, b1, d2m, d2c, d2p, b2, w3t, b3, w1at, w1bt, bs1,
      ds2, bs2, ws3t, bs3, wi, wctx)


_NC = 2
_NS = 16
_NW = _NC * _NS
_APW = NANCH // _NW
_LANES = 16
_NCH = OUT_CH // _LANES


_NBUF = 4
_WCHUNK = NTAP * _LANES


def _sc_body(table_hbm, idx_hbm, wgtb_hbm, bias_hbm, out_hbm,
             idx_v, wsp_v, rows_v, out_v, bias_v, tab_sh, rsems, wsems):
    wid = lax.axis_index("s") * _NC + lax.axis_index("c")
    base = wid * _APW
    pltpu.sync_copy(idx_hbm.at[pl.ds(base, _APW)], idx_v)
    pltpu.sync_copy(bias_hbm, bias_v)

    @pl.when(lax.axis_index("s") == 0)
    def _():
        pltpu.sync_copy(table_hbm, tab_sh)
    plsc.subcore_barrier()

    def start(a, b):
        pltpu.async_copy(tab_sh.at[idx_v.at[a]], rows_v.at[b], rsems[b])
        pltpu.async_copy(wgtb_hbm.at[pl.ds((base + a) * _WCHUNK, _WCHUNK)],
                         wsp_v.at[b], wsems[b])

    for b in range(_NBUF):
        start(b, b)

    def outer(it, carry):
        for b in range(_NBUF):
            a = it * _NBUF + b
            pltpu.make_async_copy(table_hbm.at[idx_v.at[a]], rows_v.at[b],
                                  rsems[b]).wait()
            pltpu.make_async_copy(wgtb_hbm.at[pl.ds((base + a) * _WCHUNK, _WCHUNK)],
                                  wsp_v.at[b], wsems[b]).wait()
            acc0 = tuple(bias_v[pl.ds(c * _LANES, _LANES)] for c in range(_NCH))

            def accum(j, acc):
                w = wsp_v[b, pl.ds(j * _LANES, _LANES)]
                return tuple(
                    acc[c] + w * rows_v[b, j, pl.ds(c * _LANES, _LANES)]
                    for c in range(_NCH))

            acc = plsc.parallel_loop(0, NTAP, unroll=12, carry=acc0)(accum)
            zero = jnp.zeros((_LANES,), jnp.float32)
            for c in range(_NCH):
                out_v[a, pl.ds(c * _LANES, _LANES)] = jnp.maximum(acc[c], zero)

            nxt = a + _NBUF
            @pl.when(nxt < _APW)
            def _():
                start(nxt, b)
        return carry

    lax.fori_loop(0, _APW // _NBUF, outer, 0)
    pltpu.sync_copy(out_v, out_hbm.at[pl.ds(base, _APW)])


@functools.cache
def _sc_align_fn():
    return pl.kernel(
        _sc_body,
        out_type=jax.ShapeDtypeStruct((NANCH, OUT_CH), jnp.float32),
        mesh=plsc.VectorSubcoreMesh(core_axis_name="c", subcore_axis_name="s",
                                    num_cores=_NC, num_subcores=_NS),
        scratch_types=[
            pltpu.VMEM((_APW, NTAP), jnp.int32),
            pltpu.VMEM((_NBUF, _WCHUNK), jnp.float32),
            pltpu.VMEM((_NBUF, NTAP, OUT_CH), jnp.float32),
            pltpu.VMEM((_APW, OUT_CH), jnp.float32),
            pltpu.VMEM((OUT_CH,), jnp.float32),
            pltpu.VMEM_SHARED((TSCALE * NSLOT, OUT_CH), jnp.float32),
            [pltpu.SemaphoreType.DMA] * _NBUF,
            [pltpu.SemaphoreType.DMA] * _NBUF,
        ],
    )


def kernel(x, t1_w, t1_b, t2_w, t2_b, t3_w, t3_b, s1_w, s1_b,
           s2_w, s2_b, s3_w, s3_b, c_w, c_b):
    xt = x[0].T

    w1t = t1_w[:, :, 0].T
    d2m = _block_diag_dense(t2_w[:, :, 0])
    d2c = _block_diag_dense(t2_w[:, :, 1])
    d2p = _block_diag_dense(t2_w[:, :, 2])
    w3t = t3_w[:, :, 0].T
    w1at = s1_w[:, :IN_CH, 0, 0].T
    w1bt = s1_w[:, IN_CH:, 0, 0].T
    ds2 = _block_diag_dense(s2_w[:, :, 0, 0])
    ws3t = s3_w[:, :, 0, 0].T

    wc = c_w[:, :, 0, 0].reshape(OUT_CH, IN_CH, NSLOT)
    wi = wc[:, :, :ROI_SIZE].transpose(1, 2, 0).reshape(IN_CH, ROI_SIZE * OUT_CH)
    wctx = wc[:, :, ROI_SIZE:].transpose(1, 2, 0).reshape(IN_CH, CTX_SIZE * OUT_CH)

    table = _tc_table(
        xt, w1t, t1_b[None, :], d2m, d2c, d2p, t2_b[None, :], w3t,
        t3_b[None, :], w1at, w1bt, s1_b[None, :], ds2, s2_b[None, :],
        ws3t, s3_b[None, :], wi, wctx,
    ).reshape(TSCALE * NSLOT, OUT_CH)

    out = _sc_align_fn()(table, jnp.asarray(_IDX_NP), jnp.asarray(_WGTB_NP), c_b)
    return out.reshape(DSCALE, TSCALE, OUT_CH).transpose(2, 0, 1)[None]
